# Initial kernel scaffold; baseline (speedup 1.0000x reference)
import jax
import jax.numpy as jnp
from jax.experimental import pallas as pl


def kernel(src_seqs, C):
    raise NotImplementedError("write your pallas kernel here")


# speedup vs baseline: 5.9680x; 5.9680x over previous
import functools

import jax
import jax.numpy as jnp
from jax import lax
from jax.experimental import pallas as pl
from jax.experimental.pallas import tpu as pltpu
from jax.experimental.pallas import tpu_sc as plsc

VOCAB = 100000
DIM = 128
HOPS = 3
B = 1024
L = 50
M = 6

NC = 2
NS = 16
NW = NC * NS
R_TOT = HOPS * B * L
R_W = R_TOT // NW
CH_OUT = 64
CH_IDX = CH_OUT * M
IDX_COLS = 128
IDX_ROWS_PER_CH = CH_IDX // IDX_COLS
N_CH = R_W // CH_OUT
LANES = 16


def _sc_gather_pool(c_flat, idx1d):
  mesh = plsc.VectorSubcoreMesh(core_axis_name="c", subcore_axis_name="s")

  @functools.partial(
      pl.kernel,
      mesh=mesh,
      out_type=jax.ShapeDtypeStruct((R_TOT, DIM), jnp.float32),
      scratch_types=[
          pltpu.VMEM((CH_IDX,), jnp.int32),
          pltpu.VMEM((CH_IDX, DIM), jnp.float32),
          pltpu.VMEM((CH_OUT, DIM), jnp.float32),
          pltpu.SemaphoreType.DMA,
      ],
  )
  def k(c_hbm, idx_hbm, m_hbm, idx_v, rows_v, out_v, sem):
    wid = lax.axis_index("s") * NC + lax.axis_index("c")
    row0 = wid * R_W
    idx0 = wid * (R_W * M)

    def chunk(c, carry):
      pltpu.sync_copy(idx_hbm.at[pl.ds(idx0 + c * CH_IDX, CH_IDX)], idx_v)
      for j in range(IDX_ROWS_PER_CH):
        pltpu.async_copy(
            c_hbm.at[idx_v.at[pl.ds(j * IDX_COLS, IDX_COLS)]],
            rows_v.at[pl.ds(j * IDX_COLS, IDX_COLS)],
            sem).wait()

      def pool(g, inner):
        base = g * M
        for d in range(DIM // LANES):
          sl = pl.ds(d * LANES, LANES)
          acc = rows_v[base, sl]
          for j in range(1, M):
            acc = acc + rows_v[base + j, sl]
          out_v[g, sl] = acc
        return inner

      lax.fori_loop(0, CH_OUT, pool, 0)
      pltpu.sync_copy(out_v, m_hbm.at[pl.ds(row0 + c * CH_OUT, CH_OUT)])
      return carry

    lax.fori_loop(0, N_CH, chunk, 0)

  return k(c_flat, idx1d)


BB = 64


def _tc_body(m_ref, sig_ref, u_ref):
  m1 = m_ref[0]
  m2 = m_ref[1]
  m3 = m_ref[2]
  u = jnp.mean(m1, axis=1)
  for ma, mc in ((m1, m2), (m2, m3)):
    logits = jnp.sum(ma * u[:, None, :], axis=2)
    p = jax.nn.softmax(logits, axis=1)
    u = u + jnp.sum(mc * p[:, :, None], axis=1)
  sig_ref[...] = jax.nn.sigmoid(m3)
  u_ref[...] = u


def _tc_recurrence(m, interpret=False):
  return pl.pallas_call(
      _tc_body,
      grid=(B // BB,),
      in_specs=[pl.BlockSpec((HOPS, BB, L, DIM), lambda i: (0, i, 0, 0))],
      out_specs=[pl.BlockSpec((BB, L, DIM), lambda i: (i, 0, 0)),
                 pl.BlockSpec((BB, DIM), lambda i: (i, 0))],
      out_shape=[jax.ShapeDtypeStruct((B, L, DIM), jnp.float32),
                 jax.ShapeDtypeStruct((B, DIM), jnp.float32)],
      interpret=interpret,
  )(m)


def kernel(src_seqs, C):
  flat = src_seqs.reshape(-1).astype(jnp.int32)
  offs = (jnp.arange(1, HOPS + 1, dtype=jnp.int32) * VOCAB)[:, None]
  idx1d = (flat[None, :] + offs).reshape(-1)
  c_flat = C.reshape((HOPS + 1) * VOCAB, DIM)
  m = _sc_gather_pool(c_flat, idx1d)
  m = m.reshape(HOPS, B, L, DIM)
  sig, u = _tc_recurrence(m)
  return (sig, u[None])


# speedup vs baseline: 9.1474x; 1.5327x over previous
import functools

import jax
import jax.numpy as jnp
from jax import lax
from jax.experimental import pallas as pl
from jax.experimental.pallas import tpu as pltpu
from jax.experimental.pallas import tpu_sc as plsc

VOCAB = 100000
DIM = 128
HOPS = 3
B = 1024
L = 50
M = 6

NC = 2
NS = 16
NW = NC * NS
R_TOT = HOPS * B * L
R_W = R_TOT // NW
CH_OUT = 48
CH_IDX = CH_OUT * M
G_SPLIT = 3
G_ROWS = CH_IDX // G_SPLIT
N_CH = R_W // CH_OUT
N_PAIR = N_CH // 2
LANES = 16


def _sc_gather_pool(c_flat, idx1d):
  mesh = plsc.VectorSubcoreMesh(core_axis_name="c", subcore_axis_name="s")

  @functools.partial(
      pl.kernel,
      mesh=mesh,
      out_type=jax.ShapeDtypeStruct((R_TOT, DIM), jnp.float32),
      scratch_types=[
          pltpu.VMEM((R_W * M,), jnp.int32),
          pltpu.VMEM((CH_IDX, DIM), jnp.float32),
          pltpu.VMEM((CH_IDX, DIM), jnp.float32),
          pltpu.VMEM((CH_OUT, DIM), jnp.float32),
          pltpu.VMEM((CH_OUT, DIM), jnp.float32),
          pltpu.SemaphoreType.DMA,
          pltpu.SemaphoreType.DMA,
      ],
  )
  def k(c_hbm, idx_hbm, m_hbm, idx_v, rows0, rows1, out0, out1, sem0, sem1):
    wid = lax.axis_index("s") * NC + lax.axis_index("c")
    row0 = wid * R_W
    idx0 = wid * (R_W * M)

    pltpu.sync_copy(idx_hbm.at[pl.ds(idx0, R_W * M)], idx_v)

    def fire(c, rows, sem):
      for j in range(G_SPLIT):
        pltpu.async_copy(
            c_hbm.at[idx_v.at[pl.ds(c * CH_IDX + j * G_ROWS, G_ROWS)]],
            rows.at[pl.ds(j * G_ROWS, G_ROWS)],
            sem)

    def drain(c, rows, sem):
      for j in range(G_SPLIT):
        pltpu.make_async_copy(
            c_hbm.at[idx_v.at[pl.ds(c * CH_IDX + j * G_ROWS, G_ROWS)]],
            rows.at[pl.ds(j * G_ROWS, G_ROWS)],
            sem).wait()

    def pool_store(c, rows, out):
      def pool(g, inner):
        base = g * M
        for d in range(DIM // LANES):
          sl = pl.ds(d * LANES, LANES)
          acc = rows[base, sl]
          for j in range(1, M):
            acc = acc + rows[base + j, sl]
          out[g, sl] = acc
        return inner

      lax.fori_loop(0, CH_OUT, pool, 0)
      pltpu.sync_copy(out, m_hbm.at[pl.ds(row0 + c * CH_OUT, CH_OUT)])

    fire(0, rows0, sem0)

    def pair(k_, carry):
      a = k_ * 2
      drain(a, rows0, sem0)
      fire(a + 1, rows1, sem1)
      pool_store(a, rows0, out0)
      drain(a + 1, rows1, sem1)

      @pl.when(k_ < N_PAIR - 1)
      def _():
        fire(a + 2, rows0, sem0)

      pool_store(a + 1, rows1, out1)
      return carry

    lax.fori_loop(0, N_PAIR, pair, 0)

  return k(c_flat, idx1d)


BB = 64


def _tc_body(m_ref, sig_ref, u_ref):
  m1 = m_ref[0]
  m2 = m_ref[1]
  m3 = m_ref[2]
  u = jnp.mean(m1, axis=1)
  for ma, mc in ((m1, m2), (m2, m3)):
    logits = jnp.sum(ma * u[:, None, :], axis=2)
    p = jax.nn.softmax(logits, axis=1)
    u = u + jnp.sum(mc * p[:, :, None], axis=1)
  sig_ref[...] = jax.nn.sigmoid(m3)
  u_ref[...] = u


def _tc_recurrence(m, interpret=False):
  return pl.pallas_call(
      _tc_body,
      grid=(B // BB,),
      in_specs=[pl.BlockSpec((HOPS, BB, L, DIM), lambda i: (0, i, 0, 0))],
      out_specs=[pl.BlockSpec((BB, L, DIM), lambda i: (i, 0, 0)),
                 pl.BlockSpec((BB, DIM), lambda i: (i, 0))],
      out_shape=[jax.ShapeDtypeStruct((B, L, DIM), jnp.float32),
                 jax.ShapeDtypeStruct((B, DIM), jnp.float32)],
      interpret=interpret,
  )(m)


def kernel(src_seqs, C):
  flat = src_seqs.reshape(-1).astype(jnp.int32)
  offs = (jnp.arange(1, HOPS + 1, dtype=jnp.int32) * VOCAB)[:, None]
  idx1d = (flat[None, :] + offs).reshape(-1)
  c_flat = C.reshape((HOPS + 1) * VOCAB, DIM)
  m = _sc_gather_pool(c_flat, idx1d)
  m = m.reshape(HOPS, B, L, DIM)
  sig, u = _tc_recurrence(m)
  return (sig, u[None])


# speedup vs baseline: 9.5982x; 1.0493x over previous
import functools

import jax
import jax.numpy as jnp
from jax import lax
from jax.experimental import pallas as pl
from jax.experimental.pallas import tpu as pltpu
from jax.experimental.pallas import tpu_sc as plsc

VOCAB = 100000
DIM = 128
HOPS = 3
B = 1024
L = 50
M = 6

NC = 2
NS = 16
NW = NC * NS
R_TOT = HOPS * B * L
R_W = R_TOT // NW
CH_OUT = 48
CH_IDX = CH_OUT * M
G_SPLIT = 3
G_ROWS = CH_IDX // G_SPLIT
N_CH = R_W // CH_OUT
N_PAIR = N_CH // 2
LANES = 16


def _sc_gather_pool(c_flat, idx1d):
  mesh = plsc.VectorSubcoreMesh(core_axis_name="c", subcore_axis_name="s")

  @functools.partial(
      pl.kernel,
      mesh=mesh,
      out_type=jax.ShapeDtypeStruct((R_TOT, DIM), jnp.float32),
      scratch_types=[
          pltpu.VMEM((R_W * M,), jnp.int32),
          pltpu.VMEM((CH_IDX, DIM), jnp.float32),
          pltpu.VMEM((CH_IDX, DIM), jnp.float32),
          pltpu.VMEM((CH_OUT, DIM), jnp.float32),
          pltpu.VMEM((CH_OUT, DIM), jnp.float32),
          pltpu.SemaphoreType.DMA,
          pltpu.SemaphoreType.DMA,
          pltpu.SemaphoreType.DMA,
          pltpu.SemaphoreType.DMA,
      ],
  )
  def k(c_hbm, idx_hbm, m_hbm, idx_v, rows0, rows1, out0, out1,
        sem0, sem1, ssem0, ssem1):
    wid = lax.axis_index("s") * NC + lax.axis_index("c")
    row0 = wid * R_W
    idx0 = wid * (R_W * M)

    pltpu.sync_copy(idx_hbm.at[pl.ds(idx0, R_W * M)], idx_v)

    def fire(c, rows, sem):
      for j in range(G_SPLIT):
        pltpu.async_copy(
            c_hbm.at[idx_v.at[pl.ds(c * CH_IDX + j * G_ROWS, G_ROWS)]],
            rows.at[pl.ds(j * G_ROWS, G_ROWS)],
            sem)

    def drain(c, rows, sem):
      for j in range(G_SPLIT):
        pltpu.make_async_copy(
            c_hbm.at[idx_v.at[pl.ds(c * CH_IDX + j * G_ROWS, G_ROWS)]],
            rows.at[pl.ds(j * G_ROWS, G_ROWS)],
            sem).wait()

    def pool(rows, out):
      def body(g2, inner):
        for h in range(2):
          g = g2 * 2 + h
          base = g * M
          for d in range(DIM // LANES):
            sl = pl.ds(d * LANES, LANES)
            acc = rows[base, sl]
            for j in range(1, M):
              acc = acc + rows[base + j, sl]
            out[g, sl] = acc
        return inner

      lax.fori_loop(0, CH_OUT // 2, body, 0)

    def fire_store(c, out, ssem):
      pltpu.async_copy(out, m_hbm.at[pl.ds(row0 + c * CH_OUT, CH_OUT)], ssem)

    def drain_store(c, out, ssem):
      pltpu.make_async_copy(
          out, m_hbm.at[pl.ds(row0 + c * CH_OUT, CH_OUT)], ssem).wait()

    fire(0, rows0, sem0)

    def pair(k_, carry):
      a = k_ * 2
      drain(a, rows0, sem0)
      fire(a + 1, rows1, sem1)

      @pl.when(k_ > 0)
      def _():
        drain_store(a - 2, out0, ssem0)

      pool(rows0, out0)
      fire_store(a, out0, ssem0)
      drain(a + 1, rows1, sem1)

      @pl.when(k_ < N_PAIR - 1)
      def _():
        fire(a + 2, rows0, sem0)

      @pl.when(k_ > 0)
      def _():
        drain_store(a - 1, out1, ssem1)

      pool(rows1, out1)
      fire_store(a + 1, out1, ssem1)
      return carry

    lax.fori_loop(0, N_PAIR, pair, 0)
    drain_store(N_CH - 2, out0, ssem0)
    drain_store(N_CH - 1, out1, ssem1)

  return k(c_flat, idx1d)


BB = 128


def _tc_body(m_ref, sig_ref, u_ref):
  m1 = m_ref[0]
  m2 = m_ref[1]
  m3 = m_ref[2]
  u = jnp.mean(m1, axis=1)
  for ma, mc in ((m1, m2), (m2, m3)):
    logits = jnp.sum(ma * u[:, None, :], axis=2)
    p = jax.nn.softmax(logits, axis=1)
    u = u + jnp.sum(mc * p[:, :, None], axis=1)
  sig_ref[...] = jax.nn.sigmoid(m3)
  u_ref[...] = u


def _tc_recurrence(m, interpret=False):
  return pl.pallas_call(
      _tc_body,
      grid=(B // BB,),
      in_specs=[pl.BlockSpec((HOPS, BB, L, DIM), lambda i: (0, i, 0, 0))],
      out_specs=[pl.BlockSpec((BB, L, DIM), lambda i: (i, 0, 0)),
                 pl.BlockSpec((BB, DIM), lambda i: (i, 0))],
      out_shape=[jax.ShapeDtypeStruct((B, L, DIM), jnp.float32),
                 jax.ShapeDtypeStruct((B, DIM), jnp.float32)],
      interpret=interpret,
  )(m)


def kernel(src_seqs, C):
  flat = src_seqs.reshape(-1).astype(jnp.int32)
  offs = (jnp.arange(1, HOPS + 1, dtype=jnp.int32) * VOCAB)[:, None]
  idx1d = (flat[None, :] + offs).reshape(-1)
  c_flat = C.reshape((HOPS + 1) * VOCAB, DIM)
  m = _sc_gather_pool(c_flat, idx1d)
  m = m.reshape(HOPS, B, L, DIM)
  sig, u = _tc_recurrence(m)
  return (sig, u[None])
